# baseline (speedup 1.0000x reference)
import jax
import jax.numpy as jnp
from jax.experimental import pallas as pl
from jax.experimental.pallas import tpu as pltpu

_SIZE = 50
_BF = 8
_BP = 32
_BN = 96
_D = 512
_S = 10


def _body(idx_ref, pos_ref, neg_ref, out_ref):
    sb = pl.program_id(1)
    idx = idx_ref[0]
    out_ref[...] = jnp.zeros(out_ref.shape, out_ref.dtype)

    @pl.when(sb == idx // _S)
    def _():
        j = idx % _S
        out_ref[0, j, :, :_BP, :] = pos_ref[0]
        out_ref[0, j, :, _BP:, :] = neg_ref[0]


def kernel(pos_save1, pos_save2, neg_save1, neg_save2, index, frame_id,
           r_pos_memory, r_neg_memory, t_pos_memory, t_neg_memory):
    del frame_id, r_pos_memory, r_neg_memory, t_pos_memory, t_neg_memory
    pos = jnp.stack([pos_save1.reshape(_BF, _BP, _D),
                     pos_save2.reshape(_BF, _BP, _D)])
    neg = jnp.stack([neg_save1.reshape(_BF, _BN, _D),
                     neg_save2.reshape(_BF, _BN, _D)])
    idx = jnp.asarray(index, jnp.int32).reshape((1,))
    grid_spec = pltpu.PrefetchScalarGridSpec(
        num_scalar_prefetch=1,
        grid=(2, _SIZE // _S),
        in_specs=[
            pl.BlockSpec((1, _BF, _BP, _D), lambda m, s, idx_ref: (m, 0, 0, 0)),
            pl.BlockSpec((1, _BF, _BN, _D), lambda m, s, idx_ref: (m, 0, 0, 0)),
        ],
        out_specs=pl.BlockSpec((1, _S, _BF, _BP + _BN, _D),
                               lambda m, s, idx_ref: (m, s, 0, 0, 0)),
    )
    return pl.pallas_call(
        _body,
        grid_spec=grid_spec,
        out_shape=jax.ShapeDtypeStruct((2, _SIZE, _BF, _BP + _BN, _D),
                                       jnp.float32),
    )(idx, pos, neg)
